# Initial kernel scaffold; baseline (speedup 1.0000x reference)
import jax
import jax.numpy as jnp
from jax.experimental import pallas as pl


def kernel(x, w0, b0, w1, b1):
    raise NotImplementedError("write your pallas kernel here")


# speedup vs baseline: 1.1922x; 1.1922x over previous
import functools

import jax
import jax.numpy as jnp
from jax.experimental import pallas as pl
from jax.experimental.pallas import tpu as pltpu

_VMEM_LIMIT = int(0.9 * 64 * 1024 * 1024)


def _mlp_kernel(x_ref, w0_ref, b0_ref, w1_ref, b1_ref, o_ref):
    h = x_ref[...].astype(jnp.bfloat16)
    y = jnp.dot(h, w0_ref[...], preferred_element_type=jnp.float32)
    y = y + b0_ref[...]
    y = y * jax.nn.sigmoid(y)
    h2 = y.astype(jnp.bfloat16)
    z = jnp.dot(h2, w1_ref[...], preferred_element_type=jnp.float32)
    o_ref[...] = z + b1_ref[...]


def kernel(x, w0, b0, w1, b1, *, tm=256):
    B, d_in = x.shape
    d_in2, d_h = w0.shape
    d_h2, d_out = w1.shape
    assert d_in == d_in2 and d_h == d_h2
    assert B % tm == 0

    const = lambda i: (0, 0)
    wkw = {"pipeline_mode": pl.Buffered(1)}
    return pl.pallas_call(
        _mlp_kernel,
        out_shape=jax.ShapeDtypeStruct((B, d_out), x.dtype),
        grid=(B // tm,),
        in_specs=[
            pl.BlockSpec((tm, d_in), lambda i: (i, 0)),
            pl.BlockSpec((d_in, d_h), const, **wkw),
            pl.BlockSpec((1, d_h), const, **wkw),
            pl.BlockSpec((d_h, d_out), const, **wkw),
            pl.BlockSpec((1, d_out), const, **wkw),
        ],
        out_specs=pl.BlockSpec((tm, d_out), lambda i: (i, 0)),
        compiler_params=pltpu.CompilerParams(
            dimension_semantics=("parallel",),
            vmem_limit_bytes=_VMEM_LIMIT,
        ),
    )(x, w0, b0, w1, b1)


# speedup vs baseline: 1.2537x; 1.0516x over previous
import functools

import jax
import jax.numpy as jnp
from jax.experimental import pallas as pl
from jax.experimental.pallas import tpu as pltpu

_VMEM_LIMIT = int(0.9 * 64 * 1024 * 1024)


def _mlp_kernel(x_ref, w0_ref, b0_ref, w1_ref, b1_ref, o_ref):
    h = x_ref[...].astype(jnp.bfloat16)
    y = jnp.dot(h, w0_ref[...], preferred_element_type=jnp.float32)
    y = y + b0_ref[...]
    y = y * jax.nn.sigmoid(y)
    h2 = y.astype(jnp.bfloat16)
    z = jnp.dot(h2, w1_ref[...], preferred_element_type=jnp.float32)
    o_ref[...] = z + b1_ref[...]


def kernel(x, w0, b0, w1, b1, *, tm=512):
    B, d_in = x.shape
    d_in2, d_h = w0.shape
    d_h2, d_out = w1.shape
    assert d_in == d_in2 and d_h == d_h2
    assert B % tm == 0

    const = lambda i: (0, 0)
    wkw = {"pipeline_mode": pl.Buffered(1)}
    return pl.pallas_call(
        _mlp_kernel,
        out_shape=jax.ShapeDtypeStruct((B, d_out), x.dtype),
        grid=(B // tm,),
        in_specs=[
            pl.BlockSpec((tm, d_in), lambda i: (i, 0)),
            pl.BlockSpec((d_in, d_h), const, **wkw),
            pl.BlockSpec((1, d_h), const, **wkw),
            pl.BlockSpec((d_h, d_out), const, **wkw),
            pl.BlockSpec((1, d_out), const, **wkw),
        ],
        out_specs=pl.BlockSpec((tm, d_out), lambda i: (i, 0)),
        compiler_params=pltpu.CompilerParams(
            dimension_semantics=("parallel",),
            vmem_limit_bytes=_VMEM_LIMIT,
        ),
    )(x, w0, b0, w1, b1)


# speedup vs baseline: 1.2692x; 1.0123x over previous
import functools

import jax
import jax.numpy as jnp
from jax.experimental import pallas as pl
from jax.experimental.pallas import tpu as pltpu

_VMEM_LIMIT = int(0.9 * 64 * 1024 * 1024)


def _mlp_kernel(x_ref, w0_ref, b0_ref, w1_ref, b1_ref, o_ref):
    h = x_ref[...].astype(jnp.bfloat16)
    y = jnp.dot(h, w0_ref[...], preferred_element_type=jnp.float32)
    y = y + b0_ref[...]
    y = y * jax.nn.sigmoid(y)
    h2 = y.astype(jnp.bfloat16)
    z = jnp.dot(h2, w1_ref[...], preferred_element_type=jnp.float32)
    o_ref[...] = z + b1_ref[...]


def kernel(x, w0, b0, w1, b1, *, tm=1024):
    B, d_in = x.shape
    d_in2, d_h = w0.shape
    d_h2, d_out = w1.shape
    assert d_in == d_in2 and d_h == d_h2
    assert B % tm == 0

    const = lambda i: (0, 0)
    wkw = {"pipeline_mode": pl.Buffered(1)}
    return pl.pallas_call(
        _mlp_kernel,
        out_shape=jax.ShapeDtypeStruct((B, d_out), x.dtype),
        grid=(B // tm,),
        in_specs=[
            pl.BlockSpec((tm, d_in), lambda i: (i, 0)),
            pl.BlockSpec((d_in, d_h), const, **wkw),
            pl.BlockSpec((1, d_h), const, **wkw),
            pl.BlockSpec((d_h, d_out), const, **wkw),
            pl.BlockSpec((1, d_out), const, **wkw),
        ],
        out_specs=pl.BlockSpec((tm, d_out), lambda i: (i, 0)),
        compiler_params=pltpu.CompilerParams(
            dimension_semantics=("parallel",),
            vmem_limit_bytes=_VMEM_LIMIT,
        ),
    )(x, w0, b0, w1, b1)
